# Initial kernel scaffold; baseline (speedup 1.0000x reference)
import jax
import jax.numpy as jnp
from jax.experimental import pallas as pl


def kernel(x, edge_index, batch_ids, Wl0, Wr0, b0, Wl1, Wr1, b1, Wl2, Wr2, b2, Wmax, bmax, Wfc1, bfc1, Wfc2, bfc2):
    raise NotImplementedError("write your pallas kernel here")


# speedup vs baseline: 1.0207x; 1.0207x over previous
import functools

import jax
import jax.numpy as jnp
from jax.experimental import pallas as pl
from jax.experimental.pallas import tpu as pltpu

N = 10000
NP = 10240
E = 320000
D = 128
H = 128
B = 16
BLK = 512
NBLK = NP // BLK


def _dense_body(agg_ref, h_ref, wl_ref, wr_ref, b_ref, wmax_ref, bmax_ref, out_ref):
    s = (jnp.dot(agg_ref[...], wl_ref[...], preferred_element_type=jnp.float32)
         + jnp.dot(h_ref[...], wr_ref[...], preferred_element_type=jnp.float32)
         + b_ref[...])
    s = jnp.maximum(s, 0.0)
    o = jnp.dot(s, wmax_ref[...], preferred_element_type=jnp.float32) + bmax_ref[...]
    out_ref[...] = jnp.maximum(o, 0.0)


_row_spec = pl.BlockSpec((BLK, H), lambda i: (i, 0))
_full128 = pl.BlockSpec((H, H), lambda i: (0, 0))
_bias_spec = pl.BlockSpec((1, H), lambda i: (0, 0))

_dense_layer = pl.pallas_call(
    _dense_body,
    out_shape=jax.ShapeDtypeStruct((NP, H), jnp.float32),
    grid=(NBLK,),
    in_specs=[_row_spec, _row_spec, _full128, _full128, _bias_spec, _full128, _bias_spec],
    out_specs=_row_spec,
)


def _readout_body(h1_ref, h2_ref, h3_ref, oh_ref, w1a_ref, w1b_ref, w1c_ref,
                  bfc1_ref, wfc2_ref, bfc2_ref, out_ref,
                  acc1, acc2, acc3, cnt):
    i = pl.program_id(0)

    @pl.when(i == 0)
    def _init():
        acc1[...] = jnp.zeros_like(acc1)
        acc2[...] = jnp.zeros_like(acc2)
        acc3[...] = jnp.zeros_like(acc3)
        cnt[...] = jnp.zeros_like(cnt)

    oht = oh_ref[...].T
    acc1[...] += jnp.dot(oht, h1_ref[...], preferred_element_type=jnp.float32)
    acc2[...] += jnp.dot(oht, h2_ref[...], preferred_element_type=jnp.float32)
    acc3[...] += jnp.dot(oht, h3_ref[...], preferred_element_type=jnp.float32)
    cnt[...] += jnp.sum(oh_ref[...], axis=0, keepdims=True)

    @pl.when(i == pl.num_programs(0) - 1)
    def _fin():
        c = jnp.maximum(cnt[...], 1.0).T
        z = (jnp.dot(acc1[...] / c, w1a_ref[...], preferred_element_type=jnp.float32)
             + jnp.dot(acc2[...] / c, w1b_ref[...], preferred_element_type=jnp.float32)
             + jnp.dot(acc3[...] / c, w1c_ref[...], preferred_element_type=jnp.float32)
             + bfc1_ref[...])
        z = jnp.maximum(z, 0.0)
        o = jnp.dot(z, wfc2_ref[...], preferred_element_type=jnp.float32) + bfc2_ref[...]
        col = jax.lax.broadcasted_iota(jnp.int32, o.shape, 1)
        o = jnp.where(col < 2, o, -jnp.inf)
        m = jnp.max(o, axis=1, keepdims=True)
        e = jnp.exp(o - m)
        out_ref[...] = e / jnp.sum(e, axis=1, keepdims=True)


_readout = pl.pallas_call(
    _readout_body,
    out_shape=jax.ShapeDtypeStruct((B, H), jnp.float32),
    grid=(NBLK,),
    in_specs=[_row_spec, _row_spec, _row_spec,
              pl.BlockSpec((BLK, B), lambda i: (i, 0)),
              _full128, _full128, _full128, _bias_spec, _full128, _bias_spec],
    out_specs=pl.BlockSpec((B, H), lambda i: (0, 0)),
    scratch_shapes=[pltpu.VMEM((B, H), jnp.float32)] * 3 + [pltpu.VMEM((1, B), jnp.float32)],
)


def kernel(x, edge_index, batch_ids, Wl0, Wr0, b0, Wl1, Wr1, b1, Wl2, Wr2, b2,
           Wmax, bmax, Wfc1, bfc1, Wfc2, bfc2):
    src = edge_index[0].astype(jnp.int32)
    dst = edge_index[1].astype(jnp.int32)
    bid = batch_ids.astype(jnp.int32)

    xp = jnp.zeros((NP, D), jnp.float32).at[:N].set(x)
    onehot = (jnp.pad(bid, (0, NP - N), constant_values=-1)[:, None]
              == jnp.arange(B, dtype=jnp.int32)[None, :]).astype(jnp.float32)

    b0r = b0.reshape(1, H)
    b1r = b1.reshape(1, H)
    b2r = b2.reshape(1, H)
    bmaxr = bmax.reshape(1, H)
    bfc1r = bfc1.reshape(1, H)
    wfc2p = jnp.zeros((H, H), jnp.float32).at[:, :2].set(Wfc2)
    bfc2p = jnp.zeros((1, H), jnp.float32).at[0, :2].set(bfc2)

    def seg_max(h):
        agg = jax.ops.segment_max(h[:N][src], dst, num_segments=N)
        agg = jnp.where(jnp.isfinite(agg), agg, 0.0)
        return jnp.zeros((NP, H), jnp.float32).at[:N].set(agg)

    h = xp
    outs = []
    for (Wl, Wr, br) in [(Wl0, Wr0, b0r), (Wl1, Wr1, b1r), (Wl2, Wr2, b2r)]:
        agg = seg_max(h)
        h = _dense_layer(agg, h, Wl, Wr, br, Wmax, bmaxr)
        outs.append(h)

    out = _readout(outs[0], outs[1], outs[2], onehot,
                   Wfc1[:H], Wfc1[H:2 * H], Wfc1[2 * H:], bfc1r, wfc2p, bfc2p)
    return out[:, :2]


# speedup vs baseline: 1.6474x; 1.6141x over previous
import functools

import jax
import jax.numpy as jnp
from jax import lax
from jax.experimental import pallas as pl
from jax.experimental.pallas import tpu as pltpu
from jax.experimental.pallas import tpu_sc as plsc

N = 10000
NP = 10240
E = 320000
D = 128
H = 128
B = 16
BLK = 512
NBLK = NP // BLK

NTILES = 32
ROWS_PER_TILE = NP // NTILES
ACC_ROWS = 512
CH = 1280
NCHUNK = E // CH
FL = 2048
RING = FL + CH
EROW = E + FL
G = 128
PACKED_PAD = ROWS_PER_TILE

_sc_mesh = plsc.VectorSubcoreMesh(core_axis_name="c", subcore_axis_name="s")


def _wid():
    return pl.multiple_of(
        (lax.axis_index("s") * 2 + lax.axis_index("c")) * 1, 1)


def _extract(vec, lane):
    m = lax.iota(jnp.int32, 16) == lane
    return jnp.max(jnp.where(m, vec, 0))


@functools.partial(
    pl.kernel,
    out_type=(jax.ShapeDtypeStruct((NTILES * EROW,), jnp.int32),
              jax.ShapeDtypeStruct((NTILES * 16,), jnp.int32)),
    mesh=_sc_mesh,
    compiler_params=pltpu.CompilerParams(needs_layout_passes=False),
    scratch_types=[
        pltpu.VMEM((CH,), jnp.int32),
        pltpu.VMEM((CH,), jnp.int32),
        pltpu.VMEM((RING + 16,), jnp.int32),
        pltpu.VMEM((16,), jnp.int32),
    ],
)
def _sc_bucket(src_hbm, dst_hbm, packed_hbm, counts_hbm, srcb, dstb, ring, cntb):
    wid = _wid()
    lo = wid * ROWS_PER_TILE
    iota = lax.iota(jnp.int32, 16)
    pad16 = jnp.full((16,), PACKED_PAD, jnp.int32)

    for i in range(0, RING + 16, 16):
        ring[pl.ds(i, 16)] = pad16

    def chunk_body(c, carry):
        cursor, base = carry
        pltpu.sync_copy(src_hbm.at[pl.ds(pl.multiple_of(c * CH, 8), CH)], srcb)
        pltpu.sync_copy(dst_hbm.at[pl.ds(pl.multiple_of(c * CH, 8), CH)], dstb)

        def group(g, cur):
            off = g * 16
            s16 = plsc.load_gather(srcb, [off + iota])
            d16 = plsc.load_gather(dstb, [off + iota])
            dl = d16 - lo
            m = (dl >= 0) & (dl < ROWS_PER_TILE)
            packed = jnp.where(m, (s16 << 9) | dl, PACKED_PAD)
            mi = m.astype(jnp.int32)
            pref = plsc.cumsum(mi)
            pos = jnp.where(m, cur + (pref - mi), RING + iota)
            plsc.store_scatter(ring, [pos], packed)
            return cur + jnp.max(pref)

        cursor = lax.fori_loop(0, CH // 16, group, cursor)

        def flush(args):
            cur, bs = args
            pltpu.sync_copy(ring.at[pl.ds(0, FL)],
                            packed_hbm.at[pl.ds(pl.multiple_of(wid * EROW + bs, 8), FL)])
            for i in range(0, CH, 16):
                ring[pl.ds(i, 16)] = ring[pl.ds(FL + i, 16)]
            return cur - FL, bs + FL

        cursor, base = lax.cond(cursor >= FL, flush, lambda a: a, (cursor, base))
        return cursor, base

    cursor, base = lax.fori_loop(0, NCHUNK, chunk_body, (jnp.int32(0), jnp.int32(0)))
    pltpu.sync_copy(ring.at[pl.ds(0, FL)],
                    packed_hbm.at[pl.ds(pl.multiple_of(wid * EROW + base, 8), FL)])
    cntb[...] = jnp.broadcast_to(base + cursor, (16,)).astype(jnp.int32)
    pltpu.sync_copy(cntb, counts_hbm.at[pl.ds(pl.multiple_of(wid * 16, 8), 16)])


@functools.partial(
    pl.kernel,
    out_type=jax.ShapeDtypeStruct((NP, H), jnp.float32),
    mesh=_sc_mesh,
    compiler_params=pltpu.CompilerParams(needs_layout_passes=False),
    scratch_types=[
        pltpu.VMEM((ACC_ROWS, H), jnp.float32),
        pltpu.VMEM((G, H), jnp.float32),
        pltpu.VMEM((G,), jnp.int32),
        pltpu.VMEM((G,), jnp.int32),
        pltpu.VMEM((16,), jnp.int32),
        pltpu.SemaphoreType.DMA,
    ],
)
def _sc_gather_max(h_hbm, packed_hbm, counts_hbm, agg_hbm, acc, rows, pbuf, idxb,
                   cntb, sem):
    wid = _wid()
    iota = lax.iota(jnp.int32, 16)
    ninf_row = jnp.full((16,), float("-inf"), jnp.float32)

    def init_row(i, _):
        for k in range(H // 16):
            acc[i, pl.ds(k * 16, 16)] = ninf_row
        return 0
    lax.fori_loop(0, ACC_ROWS, init_row, 0)

    pltpu.sync_copy(counts_hbm.at[pl.ds(pl.multiple_of(wid * 16, 8), 16)], cntb)
    cnt = jnp.max(cntb[...])
    nchunks = (cnt + (G - 1)) // G

    def chunk(g, _):
        pltpu.sync_copy(
            packed_hbm.at[pl.ds(pl.multiple_of(wid * EROW + g * G, 8), G)], pbuf)
        for j in range(G // 16):
            idxb[pl.ds(j * 16, 16)] = pbuf[pl.ds(j * 16, 16)] >> 9
        pltpu.async_copy(h_hbm.at[idxb], rows, sem).wait()

        for j in range(G // 16):
            pvec = pbuf[pl.ds(j * 16, 16)]
            for l in range(16):
                p = jnp.max(jnp.where(iota == l, pvec, 0))
                dl = p & 511
                e = j * 16 + l
                for k in range(H // 16):
                    sl = pl.ds(k * 16, 16)
                    acc[dl, sl] = jnp.maximum(acc[dl, sl], rows[e, sl])
        return 0

    lax.fori_loop(0, nchunks, chunk, 0)

    def fin_row(i, _):
        for k in range(H // 16):
            sl = pl.ds(k * 16, 16)
            v = acc[i, sl]
            acc[i, sl] = jnp.where(v > -jnp.inf, v, 0.0)
        return 0
    lax.fori_loop(0, ROWS_PER_TILE, fin_row, 0)
    pltpu.sync_copy(acc.at[pl.ds(0, ROWS_PER_TILE)],
                    agg_hbm.at[pl.ds(wid * ROWS_PER_TILE, ROWS_PER_TILE)])


def _dense_body(agg_ref, h_ref, wl_ref, wr_ref, b_ref, wmax_ref, bmax_ref, out_ref):
    s = (jnp.dot(agg_ref[...], wl_ref[...], preferred_element_type=jnp.float32)
         + jnp.dot(h_ref[...], wr_ref[...], preferred_element_type=jnp.float32)
         + b_ref[...])
    s = jnp.maximum(s, 0.0)
    o = jnp.dot(s, wmax_ref[...], preferred_element_type=jnp.float32) + bmax_ref[...]
    out_ref[...] = jnp.maximum(o, 0.0)


_row_spec = pl.BlockSpec((BLK, H), lambda i: (i, 0))
_full128 = pl.BlockSpec((H, H), lambda i: (0, 0))
_bias_spec = pl.BlockSpec((1, H), lambda i: (0, 0))

_dense_layer = pl.pallas_call(
    _dense_body,
    out_shape=jax.ShapeDtypeStruct((NP, H), jnp.float32),
    grid=(NBLK,),
    in_specs=[_row_spec, _row_spec, _full128, _full128, _bias_spec, _full128, _bias_spec],
    out_specs=_row_spec,
)


def _readout_body(h1_ref, h2_ref, h3_ref, oh_ref, w1a_ref, w1b_ref, w1c_ref,
                  bfc1_ref, wfc2_ref, bfc2_ref, out_ref,
                  acc1, acc2, acc3, cnt):
    i = pl.program_id(0)

    @pl.when(i == 0)
    def _init():
        acc1[...] = jnp.zeros_like(acc1)
        acc2[...] = jnp.zeros_like(acc2)
        acc3[...] = jnp.zeros_like(acc3)
        cnt[...] = jnp.zeros_like(cnt)

    oht = oh_ref[...].T
    acc1[...] += jnp.dot(oht, h1_ref[...], preferred_element_type=jnp.float32)
    acc2[...] += jnp.dot(oht, h2_ref[...], preferred_element_type=jnp.float32)
    acc3[...] += jnp.dot(oht, h3_ref[...], preferred_element_type=jnp.float32)
    cnt[...] += jnp.sum(oh_ref[...], axis=0, keepdims=True)

    @pl.when(i == pl.num_programs(0) - 1)
    def _fin():
        c = jnp.maximum(cnt[...], 1.0).T
        z = (jnp.dot(acc1[...] / c, w1a_ref[...], preferred_element_type=jnp.float32)
             + jnp.dot(acc2[...] / c, w1b_ref[...], preferred_element_type=jnp.float32)
             + jnp.dot(acc3[...] / c, w1c_ref[...], preferred_element_type=jnp.float32)
             + bfc1_ref[...])
        z = jnp.maximum(z, 0.0)
        o = jnp.dot(z, wfc2_ref[...], preferred_element_type=jnp.float32) + bfc2_ref[...]
        col = jax.lax.broadcasted_iota(jnp.int32, o.shape, 1)
        o = jnp.where(col < 2, o, -jnp.inf)
        m = jnp.max(o, axis=1, keepdims=True)
        e = jnp.exp(o - m)
        out_ref[...] = e / jnp.sum(e, axis=1, keepdims=True)


_readout = pl.pallas_call(
    _readout_body,
    out_shape=jax.ShapeDtypeStruct((B, H), jnp.float32),
    grid=(NBLK,),
    in_specs=[_row_spec, _row_spec, _row_spec,
              pl.BlockSpec((BLK, B), lambda i: (i, 0)),
              _full128, _full128, _full128, _bias_spec, _full128, _bias_spec],
    out_specs=pl.BlockSpec((B, H), lambda i: (0, 0)),
    scratch_shapes=[pltpu.VMEM((B, H), jnp.float32)] * 3 + [pltpu.VMEM((1, B), jnp.float32)],
)


def kernel(x, edge_index, batch_ids, Wl0, Wr0, b0, Wl1, Wr1, b1, Wl2, Wr2, b2,
           Wmax, bmax, Wfc1, bfc1, Wfc2, bfc2):
    src = edge_index[0].astype(jnp.int32)
    dst = edge_index[1].astype(jnp.int32)
    bid = batch_ids.astype(jnp.int32)

    xp = jnp.zeros((NP, D), jnp.float32).at[:N].set(x)
    onehot = (jnp.pad(bid, (0, NP - N), constant_values=-1)[:, None]
              == jnp.arange(B, dtype=jnp.int32)[None, :]).astype(jnp.float32)

    b0r = b0.reshape(1, H)
    b1r = b1.reshape(1, H)
    b2r = b2.reshape(1, H)
    bmaxr = bmax.reshape(1, H)
    bfc1r = bfc1.reshape(1, H)
    wfc2p = jnp.zeros((H, H), jnp.float32).at[:, :2].set(Wfc2)
    bfc2p = jnp.zeros((1, H), jnp.float32).at[0, :2].set(bfc2)

    packed, counts = _sc_bucket(src, dst)

    h = xp
    outs = []
    for (Wl, Wr, br) in [(Wl0, Wr0, b0r), (Wl1, Wr1, b1r), (Wl2, Wr2, b2r)]:
        agg = _sc_gather_max(h, packed, counts)
        h = _dense_layer(agg, h, Wl, Wr, br, Wmax, bmaxr)
        outs.append(h)

    out = _readout(outs[0], outs[1], outs[2], onehot,
                   Wfc1[:H], Wfc1[H:2 * H], Wfc1[2 * H:], bfc1r, wfc2p, bfc2p)
    return out[:, :2]


# speedup vs baseline: 2.2835x; 1.3861x over previous
import functools

import jax
import jax.numpy as jnp
from jax import lax
from jax.experimental import pallas as pl
from jax.experimental.pallas import tpu as pltpu
from jax.experimental.pallas import tpu_sc as plsc

N = 10000
NP = 10240
E = 320000
D = 128
H = 128
B = 16
BLK = 512
NBLK = NP // BLK

NTILES = 32
ROWS_PER_TILE = NP // NTILES
ACC_ROWS = 384
CH = 1280
NCHUNK = E // CH
FL = 2048
RING = FL + CH
EROW = E + 2 * FL
G = 128
GC = 256
PACKED_PAD = ROWS_PER_TILE

_sc_mesh = plsc.VectorSubcoreMesh(core_axis_name="c", subcore_axis_name="s")


def _wid():
    return pl.multiple_of(
        (lax.axis_index("s") * 2 + lax.axis_index("c")) * 1, 1)


def _extract(vec, lane):
    m = lax.iota(jnp.int32, 16) == lane
    return jnp.max(jnp.where(m, vec, 0))


@functools.partial(
    pl.kernel,
    out_type=(jax.ShapeDtypeStruct((NTILES * EROW,), jnp.int32),
              jax.ShapeDtypeStruct((NTILES * 16,), jnp.int32)),
    mesh=_sc_mesh,
    compiler_params=pltpu.CompilerParams(needs_layout_passes=False),
    scratch_types=[
        pltpu.VMEM((CH,), jnp.int32),
        pltpu.VMEM((CH,), jnp.int32),
        pltpu.VMEM((CH,), jnp.int32),
        pltpu.VMEM((CH,), jnp.int32),
        pltpu.VMEM((RING + 16,), jnp.int32),
        pltpu.VMEM((16,), jnp.int32),
        pltpu.SemaphoreType.DMA,
        pltpu.SemaphoreType.DMA,
        pltpu.SemaphoreType.DMA,
        pltpu.SemaphoreType.DMA,
    ],
)
def _sc_bucket(src_hbm, dst_hbm, packed_hbm, counts_hbm,
               srcA, dstA, srcB, dstB, ring, cntb, ssA, sdA, ssB, sdB):
    wid = _wid()
    lo = wid * ROWS_PER_TILE
    iota = lax.iota(jnp.int32, 16)
    pad16 = jnp.full((16,), PACKED_PAD, jnp.int32)
    NPAIR = NCHUNK // 2

    for i in range(0, RING + 16, 16):
        ring[pl.ds(i, 16)] = pad16

    def start_edges(c, sb, db, ss, sd):
        off = pl.multiple_of(c * CH, 8)
        pltpu.async_copy(src_hbm.at[pl.ds(off, CH)], sb, ss)
        pltpu.async_copy(dst_hbm.at[pl.ds(off, CH)], db, sd)

    def wait_edges(sb, db, ss, sd):
        pltpu.make_async_copy(src_hbm.at[pl.ds(0, CH)], sb, ss).wait()
        pltpu.make_async_copy(dst_hbm.at[pl.ds(0, CH)], db, sd).wait()

    def scan_chunk(sb, db, carry):
        def group(g, cur):
            off = g * 16
            s16 = plsc.load_gather(sb, [off + iota])
            d16 = plsc.load_gather(db, [off + iota])
            dl = d16 - lo
            m = (dl >= 0) & (dl < ROWS_PER_TILE)
            packed = jnp.where(m, (s16 << 9) | dl, PACKED_PAD)
            mi = m.astype(jnp.int32)
            pref = plsc.cumsum(mi)
            pos = jnp.where(m, cur + (pref - mi), RING + iota)
            plsc.store_scatter(ring, [pos], packed)
            return cur + jnp.max(pref)

        cursor = lax.fori_loop(0, CH // 16, group, carry[0])
        base = carry[1]

        def flush(args):
            cur, bs = args
            pltpu.sync_copy(ring.at[pl.ds(0, FL)],
                            packed_hbm.at[pl.ds(pl.multiple_of(wid * EROW + bs, 8), FL)])
            for i in range(0, CH, 16):
                ring[pl.ds(i, 16)] = ring[pl.ds(FL + i, 16)]
            return cur - FL, bs + FL

        return lax.cond(cursor >= FL, flush, lambda a: a, (cursor, base))

    start_edges(0, srcA, dstA, ssA, sdA)
    start_edges(1, srcB, dstB, ssB, sdB)

    def pair_body(t, carry):
        wait_edges(srcA, dstA, ssA, sdA)
        carry = scan_chunk(srcA, dstA, carry)

        @pl.when(t < NPAIR - 1)
        def _():
            start_edges(2 * t + 2, srcA, dstA, ssA, sdA)

        wait_edges(srcB, dstB, ssB, sdB)
        carry = scan_chunk(srcB, dstB, carry)

        @pl.when(t < NPAIR - 1)
        def _():
            start_edges(2 * t + 3, srcB, dstB, ssB, sdB)
        return carry

    cursor, base = lax.fori_loop(0, NPAIR, pair_body,
                                 (jnp.int32(0), jnp.int32(0)))
    pltpu.sync_copy(ring.at[pl.ds(0, FL)],
                    packed_hbm.at[pl.ds(pl.multiple_of(wid * EROW + base, 8), FL)])
    pltpu.sync_copy(ring.at[pl.ds(0, FL)],
                    packed_hbm.at[pl.ds(pl.multiple_of(wid * EROW + base + FL, 8), FL)])
    cntb[...] = jnp.broadcast_to(base + cursor, (16,)).astype(jnp.int32)
    pltpu.sync_copy(cntb, counts_hbm.at[pl.ds(pl.multiple_of(wid * 16, 8), 16)])


@functools.partial(
    pl.kernel,
    out_type=jax.ShapeDtypeStruct((NP, H), jnp.float32),
    mesh=_sc_mesh,
    compiler_params=pltpu.CompilerParams(needs_layout_passes=False),
    scratch_types=[
        pltpu.VMEM((ACC_ROWS, H), jnp.float32),
        pltpu.VMEM((GC, H), jnp.float32),
        pltpu.VMEM((GC, H), jnp.float32),
        pltpu.VMEM((GC,), jnp.int32),
        pltpu.VMEM((GC,), jnp.int32),
        pltpu.VMEM((16, 16), jnp.int32),
        pltpu.VMEM((GC,), jnp.int32),
        pltpu.VMEM((GC,), jnp.int32),
        pltpu.VMEM((16,), jnp.int32),
        pltpu.SemaphoreType.DMA,
        pltpu.SemaphoreType.DMA,
        pltpu.SemaphoreType.DMA,
        pltpu.SemaphoreType.DMA,
    ],
)
def _sc_gather_max(h_hbm, packed_hbm, counts_hbm, agg_hbm,
                   acc, rowsA, rowsB, pbA, pbB, pbT, ixA, ixB, cntb,
                   spA, spB, sgA, sgB):
    wid = _wid()
    iota = lax.iota(jnp.int32, 16)
    rowbase = wid * EROW
    ninf_row = jnp.full((16,), float("-inf"), jnp.float32)

    def init_row(i, _):
        for k in range(H // 16):
            acc[i, pl.ds(k * 16, 16)] = ninf_row
        return 0
    lax.fori_loop(0, ACC_ROWS, init_row, 0)

    pltpu.sync_copy(counts_hbm.at[pl.ds(pl.multiple_of(wid * 16, 8), 16)], cntb)
    cnt = jnp.max(cntb[...])
    nch = (cnt + (GC - 1)) // GC
    npair = jnp.maximum((nch + 1) // 2, 1)

    def start_pb(pb, sem, c):
        off = pl.multiple_of(rowbase + c * GC, 8)
        pltpu.async_copy(packed_hbm.at[pl.ds(off, GC)], pb, sem)

    def wait_pb(pb, sem):
        pltpu.make_async_copy(packed_hbm.at[pl.ds(0, GC)], pb, sem).wait()

    def ix_and_gather(pb, ix, rows, sem):
        for i in range(GC // 16):
            ix[pl.ds(i * 16, 16)] = pb[pl.ds(i * 16, 16)] >> 9
        pltpu.async_copy(h_hbm.at[ix.at[pl.ds(0, G)]], rows.at[pl.ds(0, G)], sem)
        pltpu.async_copy(h_hbm.at[ix.at[pl.ds(G, G)]], rows.at[pl.ds(G, G)], sem)

    def wait_gather(rows, sem):
        pltpu.make_async_copy(h_hbm.at[pl.ds(0, G)], rows.at[pl.ds(0, G)], sem).wait()
        pltpu.make_async_copy(h_hbm.at[pl.ds(0, G)], rows.at[pl.ds(G, G)], sem).wait()

    def copy_pbT(pb):
        for i in range(16):
            pbT[i, :] = pb[pl.ds(i * 16, 16)]

    def rmw(rows):
        def jbody(j, _):
            pvec = pbT[j, :]
            for l in range(16):
                p = jnp.max(jnp.where(iota == l, pvec, 0))
                dl = jnp.minimum(p & 511, ACC_ROWS - 1)
                e = j * 16 + l
                for k in range(H // 16):
                    sl = pl.ds(k * 16, 16)
                    acc[dl, sl] = jnp.maximum(acc[dl, sl], rows[e, sl])
            return 0
        lax.fori_loop(0, GC // 16, jbody, 0)

    start_pb(pbA, spA, 0)
    start_pb(pbB, spB, 1)
    wait_pb(pbA, spA)
    ix_and_gather(pbA, ixA, rowsA, sgA)
    wait_pb(pbB, spB)
    ix_and_gather(pbB, ixB, rowsB, sgB)

    def body(t, _):
        wait_gather(rowsA, sgA)
        copy_pbT(pbA)

        @pl.when(t < npair - 1)
        def _():
            start_pb(pbA, spA, 2 * t + 2)

        rmw(rowsA)

        @pl.when(t < npair - 1)
        def _():
            wait_pb(pbA, spA)
            ix_and_gather(pbA, ixA, rowsA, sgA)

        wait_gather(rowsB, sgB)
        copy_pbT(pbB)

        @pl.when(t < npair - 1)
        def _():
            start_pb(pbB, spB, 2 * t + 3)

        rmw(rowsB)

        @pl.when(t < npair - 1)
        def _():
            wait_pb(pbB, spB)
            ix_and_gather(pbB, ixB, rowsB, sgB)
        return 0

    lax.fori_loop(0, npair, body, 0)

    def fin_row(i, _):
        for k in range(H // 16):
            sl = pl.ds(k * 16, 16)
            v = acc[i, sl]
            acc[i, sl] = jnp.where(v > -jnp.inf, v, 0.0)
        return 0
    lax.fori_loop(0, ROWS_PER_TILE, fin_row, 0)
    pltpu.sync_copy(acc.at[pl.ds(0, ROWS_PER_TILE)],
                    agg_hbm.at[pl.ds(wid * ROWS_PER_TILE, ROWS_PER_TILE)])


def _dense_body(agg_ref, h_ref, wl_ref, wr_ref, b_ref, wmax_ref, bmax_ref, out_ref):
    s = (jnp.dot(agg_ref[...], wl_ref[...], preferred_element_type=jnp.float32)
         + jnp.dot(h_ref[...], wr_ref[...], preferred_element_type=jnp.float32)
         + b_ref[...])
    s = jnp.maximum(s, 0.0)
    o = jnp.dot(s, wmax_ref[...], preferred_element_type=jnp.float32) + bmax_ref[...]
    out_ref[...] = jnp.maximum(o, 0.0)


_row_spec = pl.BlockSpec((BLK, H), lambda i: (i, 0))
_full128 = pl.BlockSpec((H, H), lambda i: (0, 0))
_bias_spec = pl.BlockSpec((1, H), lambda i: (0, 0))

_dense_layer = pl.pallas_call(
    _dense_body,
    out_shape=jax.ShapeDtypeStruct((NP, H), jnp.float32),
    grid=(NBLK,),
    in_specs=[_row_spec, _row_spec, _full128, _full128, _bias_spec, _full128, _bias_spec],
    out_specs=_row_spec,
)


def _readout_body(h1_ref, h2_ref, h3_ref, oh_ref, w1a_ref, w1b_ref, w1c_ref,
                  bfc1_ref, wfc2_ref, bfc2_ref, out_ref,
                  acc1, acc2, acc3, cnt):
    i = pl.program_id(0)

    @pl.when(i == 0)
    def _init():
        acc1[...] = jnp.zeros_like(acc1)
        acc2[...] = jnp.zeros_like(acc2)
        acc3[...] = jnp.zeros_like(acc3)
        cnt[...] = jnp.zeros_like(cnt)

    oht = oh_ref[...].T
    acc1[...] += jnp.dot(oht, h1_ref[...], preferred_element_type=jnp.float32)
    acc2[...] += jnp.dot(oht, h2_ref[...], preferred_element_type=jnp.float32)
    acc3[...] += jnp.dot(oht, h3_ref[...], preferred_element_type=jnp.float32)
    cnt[...] += jnp.sum(oh_ref[...], axis=0, keepdims=True)

    @pl.when(i == pl.num_programs(0) - 1)
    def _fin():
        c = jnp.maximum(cnt[...], 1.0).T
        z = (jnp.dot(acc1[...] / c, w1a_ref[...], preferred_element_type=jnp.float32)
             + jnp.dot(acc2[...] / c, w1b_ref[...], preferred_element_type=jnp.float32)
             + jnp.dot(acc3[...] / c, w1c_ref[...], preferred_element_type=jnp.float32)
             + bfc1_ref[...])
        z = jnp.maximum(z, 0.0)
        o = jnp.dot(z, wfc2_ref[...], preferred_element_type=jnp.float32) + bfc2_ref[...]
        col = jax.lax.broadcasted_iota(jnp.int32, o.shape, 1)
        o = jnp.where(col < 2, o, -jnp.inf)
        m = jnp.max(o, axis=1, keepdims=True)
        e = jnp.exp(o - m)
        out_ref[...] = e / jnp.sum(e, axis=1, keepdims=True)


_readout = pl.pallas_call(
    _readout_body,
    out_shape=jax.ShapeDtypeStruct((B, H), jnp.float32),
    grid=(NBLK,),
    in_specs=[_row_spec, _row_spec, _row_spec,
              pl.BlockSpec((BLK, B), lambda i: (i, 0)),
              _full128, _full128, _full128, _bias_spec, _full128, _bias_spec],
    out_specs=pl.BlockSpec((B, H), lambda i: (0, 0)),
    scratch_shapes=[pltpu.VMEM((B, H), jnp.float32)] * 3 + [pltpu.VMEM((1, B), jnp.float32)],
)


def kernel(x, edge_index, batch_ids, Wl0, Wr0, b0, Wl1, Wr1, b1, Wl2, Wr2, b2,
           Wmax, bmax, Wfc1, bfc1, Wfc2, bfc2):
    src = edge_index[0].astype(jnp.int32)
    dst = edge_index[1].astype(jnp.int32)
    bid = batch_ids.astype(jnp.int32)

    xp = jnp.zeros((NP, D), jnp.float32).at[:N].set(x)
    onehot = (jnp.pad(bid, (0, NP - N), constant_values=-1)[:, None]
              == jnp.arange(B, dtype=jnp.int32)[None, :]).astype(jnp.float32)

    b0r = b0.reshape(1, H)
    b1r = b1.reshape(1, H)
    b2r = b2.reshape(1, H)
    bmaxr = bmax.reshape(1, H)
    bfc1r = bfc1.reshape(1, H)
    wfc2p = jnp.zeros((H, H), jnp.float32).at[:, :2].set(Wfc2)
    bfc2p = jnp.zeros((1, H), jnp.float32).at[0, :2].set(bfc2)

    packed, counts = _sc_bucket(src, dst)

    h = xp
    outs = []
    for (Wl, Wr, br) in [(Wl0, Wr0, b0r), (Wl1, Wr1, b1r), (Wl2, Wr2, b2r)]:
        agg = _sc_gather_max(h, packed, counts)
        h = _dense_layer(agg, h, Wl, Wr, br, Wmax, bmaxr)
        outs.append(h)

    out = _readout(outs[0], outs[1], outs[2], onehot,
                   Wfc1[:H], Wfc1[H:2 * H], Wfc1[2 * H:], bfc1r, wfc2p, bfc2p)
    return out[:, :2]


# speedup vs baseline: 2.8774x; 1.2601x over previous
import functools

import jax
import jax.numpy as jnp
from jax import lax
from jax.experimental import pallas as pl
from jax.experimental.pallas import tpu as pltpu
from jax.experimental.pallas import tpu_sc as plsc

N = 10000
NP = 10240
E = 320000
D = 128
H = 128
B = 16
BLK = 512
NBLK = NP // BLK

NTILES = 32
ROWS_PER_TILE = NP // NTILES
ACC_ROWS = 328
CH = 1280
NCHUNK = E // CH
FL = 2048
RING = FL + CH
EROW = E + 2 * FL
G = 128
GC = 128
PACKED_PAD = ROWS_PER_TILE

_sc_mesh = plsc.VectorSubcoreMesh(core_axis_name="c", subcore_axis_name="s")


def _wid():
    return pl.multiple_of(
        (lax.axis_index("s") * 2 + lax.axis_index("c")) * 1, 1)


def _extract(vec, lane):
    m = lax.iota(jnp.int32, 16) == lane
    return jnp.max(jnp.where(m, vec, 0))


@functools.partial(
    pl.kernel,
    out_type=(jax.ShapeDtypeStruct((NTILES * EROW,), jnp.int32),
              jax.ShapeDtypeStruct((NTILES * 16,), jnp.int32)),
    mesh=_sc_mesh,
    compiler_params=pltpu.CompilerParams(needs_layout_passes=False),
    scratch_types=[
        pltpu.VMEM((CH,), jnp.int32),
        pltpu.VMEM((CH,), jnp.int32),
        pltpu.VMEM((CH,), jnp.int32),
        pltpu.VMEM((CH,), jnp.int32),
        pltpu.VMEM((RING + 16,), jnp.int32),
        pltpu.VMEM((16,), jnp.int32),
        pltpu.SemaphoreType.DMA,
        pltpu.SemaphoreType.DMA,
        pltpu.SemaphoreType.DMA,
        pltpu.SemaphoreType.DMA,
    ],
)
def _sc_bucket(src_hbm, dst_hbm, packed_hbm, counts_hbm,
               srcA, dstA, srcB, dstB, ring, cntb, ssA, sdA, ssB, sdB):
    wid = _wid()
    lo = wid * ROWS_PER_TILE
    iota = lax.iota(jnp.int32, 16)
    pad16 = jnp.full((16,), PACKED_PAD, jnp.int32)
    NPAIR = NCHUNK // 2

    for i in range(0, RING + 16, 16):
        ring[pl.ds(i, 16)] = pad16

    def start_edges(c, sb, db, ss, sd):
        off = pl.multiple_of(c * CH, 8)
        pltpu.async_copy(src_hbm.at[pl.ds(off, CH)], sb, ss)
        pltpu.async_copy(dst_hbm.at[pl.ds(off, CH)], db, sd)

    def wait_edges(sb, db, ss, sd):
        pltpu.make_async_copy(src_hbm.at[pl.ds(0, CH)], sb, ss).wait()
        pltpu.make_async_copy(dst_hbm.at[pl.ds(0, CH)], db, sd).wait()

    def scan_chunk(sb, db, carry):
        UNROLL = 4

        def group(g, curv):
            for u in range(UNROLL):
                off = pl.multiple_of(g * (16 * UNROLL), 8) + u * 16
                s16 = sb[pl.ds(off, 16)]
                d16 = db[pl.ds(off, 16)]
                dl = d16 - lo
                m = (dl >= 0) & (dl < ROWS_PER_TILE)
                packed = jnp.where(m, (s16 << 9) | dl, PACKED_PAD)
                mi = m.astype(jnp.int32)
                pref = plsc.cumsum(mi)
                pos = jnp.where(m, curv + (pref - mi), RING + iota)
                plsc.store_scatter(ring, [pos], packed)
                curv = curv + plsc.all_reduce_population_count(m)
            return curv

        curv = lax.fori_loop(0, CH // (16 * UNROLL), group,
                             jnp.broadcast_to(carry[0], (16,)).astype(jnp.int32))
        cursor = curv[0]
        base = carry[1]

        def flush(args):
            cur, bs = args
            pltpu.sync_copy(ring.at[pl.ds(0, FL)],
                            packed_hbm.at[pl.ds(pl.multiple_of(wid * EROW + bs, 8), FL)])
            for i in range(0, CH, 16):
                ring[pl.ds(i, 16)] = ring[pl.ds(FL + i, 16)]
            return cur - FL, bs + FL

        return lax.cond(cursor >= FL, flush, lambda a: a, (cursor, base))

    start_edges(0, srcA, dstA, ssA, sdA)
    start_edges(1, srcB, dstB, ssB, sdB)

    def pair_body(t, carry):
        wait_edges(srcA, dstA, ssA, sdA)
        carry = scan_chunk(srcA, dstA, carry)

        @pl.when(t < NPAIR - 1)
        def _():
            start_edges(2 * t + 2, srcA, dstA, ssA, sdA)

        wait_edges(srcB, dstB, ssB, sdB)
        carry = scan_chunk(srcB, dstB, carry)

        @pl.when(t < NPAIR - 1)
        def _():
            start_edges(2 * t + 3, srcB, dstB, ssB, sdB)
        return carry

    cursor, base = lax.fori_loop(0, NPAIR, pair_body,
                                 (jnp.int32(0), jnp.int32(0)))
    pltpu.sync_copy(ring.at[pl.ds(0, FL)],
                    packed_hbm.at[pl.ds(pl.multiple_of(wid * EROW + base, 8), FL)])
    pltpu.sync_copy(ring.at[pl.ds(0, FL)],
                    packed_hbm.at[pl.ds(pl.multiple_of(wid * EROW + base + FL, 8), FL)])
    cntb[...] = jnp.broadcast_to(base + cursor, (16,)).astype(jnp.int32)
    pltpu.sync_copy(cntb, counts_hbm.at[pl.ds(pl.multiple_of(wid * 16, 8), 16)])


@functools.partial(
    pl.kernel,
    out_type=jax.ShapeDtypeStruct((NP, H), jnp.float32),
    mesh=_sc_mesh,
    compiler_params=pltpu.CompilerParams(needs_layout_passes=False),
    scratch_types=[
        pltpu.VMEM((ACC_ROWS, H), jnp.float32),
        pltpu.VMEM((ACC_ROWS, H), jnp.float32),
        pltpu.VMEM((GC, H), jnp.float32),
        pltpu.VMEM((GC, H), jnp.float32),
        pltpu.VMEM((GC,), jnp.int32),
        pltpu.VMEM((GC,), jnp.int32),
        pltpu.VMEM((16, 16), jnp.int32),
        pltpu.VMEM((GC,), jnp.int32),
        pltpu.VMEM((GC,), jnp.int32),
        pltpu.VMEM((16,), jnp.int32),
        pltpu.SemaphoreType.DMA,
        pltpu.SemaphoreType.DMA,
        pltpu.SemaphoreType.DMA,
        pltpu.SemaphoreType.DMA,
    ],
)
def _sc_gather_max(h_hbm, packed_hbm, counts_hbm, agg_hbm,
                   acc0, acc1, rowsA, rowsB, pbA, pbB, pbT, ixA, ixB, cntb,
                   spA, spB, sgA, sgB):
    wid = _wid()
    rowbase = wid * EROW
    ninf_row = jnp.full((16,), float("-inf"), jnp.float32)

    def init_row(i, _):
        for k in range(H // 16):
            acc0[i, pl.ds(k * 16, 16)] = ninf_row
            acc1[i, pl.ds(k * 16, 16)] = ninf_row
        return 0
    lax.fori_loop(0, ACC_ROWS, init_row, 0)

    pltpu.sync_copy(counts_hbm.at[pl.ds(pl.multiple_of(wid * 16, 8), 16)], cntb)
    cnt = jnp.max(cntb[...])
    nch = (cnt + (GC - 1)) // GC
    npair = jnp.maximum((nch + 1) // 2, 1)

    def start_pb(pb, sem, c):
        off = pl.multiple_of(rowbase + c * GC, 8)
        pltpu.async_copy(packed_hbm.at[pl.ds(off, GC)], pb, sem)

    def wait_pb(pb, sem):
        pltpu.make_async_copy(packed_hbm.at[pl.ds(0, GC)], pb, sem).wait()

    def ix_and_gather(pb, ix, rows, sem):
        for i in range(GC // 16):
            ix[pl.ds(i * 16, 16)] = pb[pl.ds(i * 16, 16)] >> 9
        pltpu.async_copy(h_hbm.at[ix], rows, sem)

    def wait_gather(rows, sem):
        pltpu.make_async_copy(h_hbm.at[pl.ds(0, GC)], rows, sem).wait()

    def copy_pbT(pb):
        for i in range(GC // 16):
            pbT[i, :] = pb[pl.ds(i * 16, 16)]

    def rmw(rows):
        def jbody(j, _):
            pvec = pbT[j, :]
            for l in range(16):
                p = pvec[l]
                dl = jnp.minimum(p & 511, ACC_ROWS - 1)
                e = j * 16 + l
                acc = acc0 if l % 2 == 0 else acc1
                for k in range(H // 16):
                    sl = pl.ds(k * 16, 16)
                    acc[dl, sl] = jnp.maximum(acc[dl, sl], rows[e, sl])
            return 0
        lax.fori_loop(0, GC // 16, jbody, 0)

    start_pb(pbA, spA, 0)
    start_pb(pbB, spB, 1)
    wait_pb(pbA, spA)
    ix_and_gather(pbA, ixA, rowsA, sgA)
    wait_pb(pbB, spB)
    ix_and_gather(pbB, ixB, rowsB, sgB)

    def body(t, _):
        wait_gather(rowsA, sgA)
        copy_pbT(pbA)

        @pl.when(t < npair - 1)
        def _():
            start_pb(pbA, spA, 2 * t + 2)

        rmw(rowsA)

        @pl.when(t < npair - 1)
        def _():
            wait_pb(pbA, spA)
            ix_and_gather(pbA, ixA, rowsA, sgA)

        wait_gather(rowsB, sgB)
        copy_pbT(pbB)

        @pl.when(t < npair - 1)
        def _():
            start_pb(pbB, spB, 2 * t + 3)

        rmw(rowsB)

        @pl.when(t < npair - 1)
        def _():
            wait_pb(pbB, spB)
            ix_and_gather(pbB, ixB, rowsB, sgB)
        return 0

    lax.fori_loop(0, npair, body, 0)

    def fin_row(i, _):
        for k in range(H // 16):
            sl = pl.ds(k * 16, 16)
            v = jnp.maximum(acc0[i, sl], acc1[i, sl])
            acc0[i, sl] = jnp.where(v > -jnp.inf, v, 0.0)
        return 0
    lax.fori_loop(0, ROWS_PER_TILE, fin_row, 0)
    pltpu.sync_copy(acc0.at[pl.ds(0, ROWS_PER_TILE)],
                    agg_hbm.at[pl.ds(wid * ROWS_PER_TILE, ROWS_PER_TILE)])


def _dense_body(agg_ref, h_ref, wl_ref, wr_ref, b_ref, wmax_ref, bmax_ref,
                out_ref):
    s = (jnp.dot(agg_ref[...], wl_ref[...], preferred_element_type=jnp.float32)
         + jnp.dot(h_ref[...], wr_ref[...], preferred_element_type=jnp.float32)
         + b_ref[...])
    s = jnp.maximum(s, 0.0)
    o = jnp.dot(s, wmax_ref[...], preferred_element_type=jnp.float32) + bmax_ref[...]
    out_ref[...] = jnp.maximum(o, 0.0)


_row_spec = pl.BlockSpec((BLK, H), lambda i: (i, 0))
_full128 = pl.BlockSpec((H, H), lambda i: (0, 0))
_bias_spec = pl.BlockSpec((1, H), lambda i: (0, 0))

_dense_layer = pl.pallas_call(
    _dense_body,
    out_shape=jax.ShapeDtypeStruct((NP, H), jnp.float32),
    grid=(NBLK,),
    in_specs=[_row_spec, _row_spec, _full128, _full128, _bias_spec, _full128, _bias_spec],
    out_specs=_row_spec,
)


def _readout_body(h1_ref, h2_ref, h3_ref, oh_ref, w1a_ref, w1b_ref, w1c_ref,
                  bfc1_ref, wfc2_ref, bfc2_ref, out_ref,
                  acc1, acc2, acc3, cnt):
    i = pl.program_id(0)

    @pl.when(i == 0)
    def _init():
        acc1[...] = jnp.zeros_like(acc1)
        acc2[...] = jnp.zeros_like(acc2)
        acc3[...] = jnp.zeros_like(acc3)
        cnt[...] = jnp.zeros_like(cnt)

    oht = oh_ref[...].T
    acc1[...] += jnp.dot(oht, h1_ref[...], preferred_element_type=jnp.float32)
    acc2[...] += jnp.dot(oht, h2_ref[...], preferred_element_type=jnp.float32)
    acc3[...] += jnp.dot(oht, h3_ref[...], preferred_element_type=jnp.float32)
    cnt[...] += jnp.sum(oh_ref[...], axis=0, keepdims=True)

    @pl.when(i == pl.num_programs(0) - 1)
    def _fin():
        c = jnp.maximum(cnt[...], 1.0).T
        z = (jnp.dot(acc1[...] / c, w1a_ref[...], preferred_element_type=jnp.float32)
             + jnp.dot(acc2[...] / c, w1b_ref[...], preferred_element_type=jnp.float32)
             + jnp.dot(acc3[...] / c, w1c_ref[...], preferred_element_type=jnp.float32)
             + bfc1_ref[...])
        z = jnp.maximum(z, 0.0)
        o = jnp.dot(z, wfc2_ref[...], preferred_element_type=jnp.float32) + bfc2_ref[...]
        col = jax.lax.broadcasted_iota(jnp.int32, o.shape, 1)
        o = jnp.where(col < 2, o, -jnp.inf)
        m = jnp.max(o, axis=1, keepdims=True)
        e = jnp.exp(o - m)
        out_ref[...] = e / jnp.sum(e, axis=1, keepdims=True)


_readout = pl.pallas_call(
    _readout_body,
    out_shape=jax.ShapeDtypeStruct((B, H), jnp.float32),
    grid=(NBLK,),
    in_specs=[_row_spec, _row_spec, _row_spec,
              pl.BlockSpec((BLK, B), lambda i: (i, 0)),
              _full128, _full128, _full128, _bias_spec, _full128, _bias_spec],
    out_specs=pl.BlockSpec((B, H), lambda i: (0, 0)),
    scratch_shapes=[pltpu.VMEM((B, H), jnp.float32)] * 3 + [pltpu.VMEM((1, B), jnp.float32)],
)


def kernel(x, edge_index, batch_ids, Wl0, Wr0, b0, Wl1, Wr1, b1, Wl2, Wr2, b2,
           Wmax, bmax, Wfc1, bfc1, Wfc2, bfc2):
    src = edge_index[0].astype(jnp.int32)
    dst = edge_index[1].astype(jnp.int32)
    bid = batch_ids.astype(jnp.int32)

    xp = jnp.zeros((NP, D), jnp.float32).at[:N].set(x)
    onehot = (jnp.pad(bid, (0, NP - N), constant_values=-1)[:, None]
              == jnp.arange(B, dtype=jnp.int32)[None, :]).astype(jnp.float32)

    b0r = b0.reshape(1, H)
    b1r = b1.reshape(1, H)
    b2r = b2.reshape(1, H)
    bmaxr = bmax.reshape(1, H)
    bfc1r = bfc1.reshape(1, H)
    wfc2p = jnp.zeros((H, H), jnp.float32).at[:, :2].set(Wfc2)
    bfc2p = jnp.zeros((1, H), jnp.float32).at[0, :2].set(bfc2)

    packed, counts = _sc_bucket(src, dst)

    h = xp
    outs = []
    for (Wl, Wr, br) in [(Wl0, Wr0, b0r), (Wl1, Wr1, b1r), (Wl2, Wr2, b2r)]:
        agg = _sc_gather_max(h, packed, counts)
        h = _dense_layer(agg, h, Wl, Wr, br, Wmax, bmaxr)
        outs.append(h)

    out = _readout(outs[0], outs[1], outs[2], onehot,
                   Wfc1[:H], Wfc1[H:2 * H], Wfc1[2 * H:], bfc1r, wfc2p, bfc2p)
    return out[:, :2]
